# baseline (speedup 1.0000x reference)
import functools

import jax
import jax.numpy as jnp
from jax.experimental import pallas as pl
from jax.experimental.pallas import tpu as pltpu


def _qloss_kernel(x_ref, wf_ref, out_ref, min_ref):
    i = pl.program_id(0)
    j = pl.program_id(1)
    nj = pl.num_programs(1)
    xb = x_ref[...]
    wfb = wf_ref[...]
    ct = jnp.dot(xb, wfb.T, preferred_element_type=jnp.float32)
    ones = jnp.ones((8, xb.shape[1]), dtype=jnp.float32)
    wsq = jax.lax.dot_general(
        ones, wfb * wfb, (((1,), (1,)), ((), ())),
        preferred_element_type=jnp.float32)[:1]
    part = jnp.min(wsq - 2.0 * ct, axis=1, keepdims=True)

    @pl.when(j == 0)
    def _init_min():
        min_ref[...] = part

    @pl.when(j != 0)
    def _acc_min():
        min_ref[...] = jnp.minimum(min_ref[...], part)

    @pl.when(j == nj - 1)
    def _finish_row_block():
        xsq = jnp.sum(xb * xb, axis=1, keepdims=True)
        d = jnp.sqrt(jnp.maximum(xsq + min_ref[...], 0.0))
        s = jnp.sum(d).reshape(1, 1)

        @pl.when(i == 0)
        def _init_out():
            out_ref[...] = s

        @pl.when(i != 0)
        def _acc_out():
            out_ref[...] += s


@jax.jit
def kernel(x, w):
    n, dim = x.shape
    wf = w.reshape(-1, w.shape[-1])
    k = wf.shape[0]
    bi = 256
    bj = 512
    total = pl.pallas_call(
        _qloss_kernel,
        grid=(n // bi, k // bj),
        in_specs=[
            pl.BlockSpec((bi, dim), lambda i, j: (i, 0)),
            pl.BlockSpec((bj, dim), lambda i, j: (j, 0)),
        ],
        out_specs=pl.BlockSpec((1, 1), lambda i, j: (0, 0)),
        out_shape=jax.ShapeDtypeStruct((1, 1), jnp.float32),
        scratch_shapes=[pltpu.VMEM((bi, 1), jnp.float32)],
    )(x, wf)
    return total[0, 0] / n


# speedup vs baseline: 4.6548x; 4.6548x over previous
import functools

import jax
import jax.numpy as jnp
from jax.experimental import pallas as pl
from jax.experimental.pallas import tpu as pltpu


def _qloss_kernel(xm2_ref, wf_ref, out_ref, wsq_ref):
    i = pl.program_id(0)
    bi = xm2_ref.shape[0]
    k, dim = wf_ref.shape
    bj = 512
    xm2 = xm2_ref[...]

    @pl.when(i == 0)
    def _compute_wsq():
        wfull = wf_ref[...]
        ones = jnp.ones((8, dim), dtype=jnp.bfloat16)
        wsq_ref[...] = jax.lax.dot_general(
            ones, wfull * wfull, (((1,), (1,)), ((), ())),
            preferred_element_type=jnp.float32)

    onesc = jnp.ones((8, dim), dtype=jnp.bfloat16)
    xsq = 0.25 * jax.lax.dot_general(
        xm2 * xm2, onesc, (((1,), (1,)), ((), ())),
        preferred_element_type=jnp.float32)[:, :1]

    def body(j, run_min):
        wc = wf_ref[pl.ds(j * bj, bj), :]
        ctm2 = jax.lax.dot_general(
            xm2, wc, (((1,), (1,)), ((), ())),
            preferred_element_type=jnp.float32)
        wsq = wsq_ref[:1, pl.ds(j * bj, bj)]
        part = jnp.min(wsq + ctm2, axis=1, keepdims=True)
        return jnp.minimum(run_min, part)

    m = jax.lax.fori_loop(
        0, k // bj, body,
        jnp.full((bi, 1), jnp.inf, dtype=jnp.float32))
    d = jnp.sqrt(jnp.maximum(xsq + m, 0.0))
    s = jnp.sum(d).reshape(1, 1)

    @pl.when(i == 0)
    def _init_out():
        out_ref[...] = s

    @pl.when(i != 0)
    def _acc_out():
        out_ref[...] += s


@jax.jit
def kernel(x, w):
    n, dim = x.shape
    wf = w.reshape(-1, w.shape[-1])
    k = wf.shape[0]
    xm2 = (-2.0 * x).astype(jnp.bfloat16)
    wfh = wf.astype(jnp.bfloat16)
    bi = 2048
    total = pl.pallas_call(
        _qloss_kernel,
        grid=(n // bi,),
        in_specs=[
            pl.BlockSpec((bi, dim), lambda i: (i, 0)),
            pl.BlockSpec((k, dim), lambda i: (0, 0)),
        ],
        out_specs=pl.BlockSpec((1, 1), lambda i: (0, 0)),
        out_shape=jax.ShapeDtypeStruct((1, 1), jnp.float32),
        scratch_shapes=[pltpu.VMEM((8, k), jnp.float32)],
    )(xm2, wfh)
    return total[0, 0] / n
